# baseline (speedup 1.0000x reference)
import functools

import jax
import jax.numpy as jnp
from jax import lax
from jax.experimental import pallas as pl
from jax.experimental.pallas import tpu as pltpu
from jax.experimental.pallas import tpu_sc as plsc

_EMBED = 4096
_E = 64
_K = 8
_ROWS = 1024
_N_ROWS = 16384
_CHUNKS = 2
_CHUNK = _N_ROWS // _CHUNKS


def _logits_topk_block(x_ref, w_ref, b_ref, wv_ref, ti_ref, idx_ref):
    x = x_ref[...]
    w = w_ref[...]
    logits = jax.lax.dot_general(
        x, w, (((1,), (1,)), ((), ())), preferred_element_type=jnp.float32
    ) + b_ref[...]

    lt = logits.T
    rows = jax.lax.broadcasted_iota(jnp.int32, lt.shape, 0)
    vals = lt
    maxes = []
    idxs = []
    for _ in range(_K):
        m = jnp.max(vals, axis=0, keepdims=True)
        a = jnp.min(jnp.where(vals == m, rows, _E), axis=0, keepdims=True)
        maxes.append(m)
        idxs.append(a)
        vals = jnp.where(rows == a, -jnp.inf, vals)

    e = [jnp.ones_like(maxes[0])] + [jnp.exp(m - maxes[0]) for m in maxes[1:]]
    denom = functools.reduce(jnp.add, e)
    ids = jnp.concatenate(idxs, axis=0)
    wv_ref[...] = jnp.concatenate(e, axis=0) / denom
    ti_ref[...] = ids
    idx_ref[...] = ids.T


def _tc_logits_topk(flat, weight, bias2d):
    rows = flat.shape[0]
    return pl.pallas_call(
        _logits_topk_block,
        grid=(rows // _ROWS,),
        in_specs=[
            pl.BlockSpec((_ROWS, _EMBED), lambda i: (i, 0)),
            pl.BlockSpec((_E, _EMBED), lambda i: (0, 0)),
            pl.BlockSpec((1, _E), lambda i: (0, 0)),
        ],
        out_specs=[
            pl.BlockSpec((_K, _ROWS), lambda i: (0, i)),
            pl.BlockSpec((_K, _ROWS), lambda i: (0, i)),
            pl.BlockSpec((_ROWS, _K), lambda i: (i, 0)),
        ],
        out_shape=[
            jax.ShapeDtypeStruct((_K, rows), jnp.float32),
            jax.ShapeDtypeStruct((_K, rows), jnp.int32),
            jax.ShapeDtypeStruct((rows, _K), jnp.int32),
        ],
    )(flat, weight, bias2d)


_SC_INFO = plsc.get_sparse_core_info()
_NW = _SC_INFO.num_cores * _SC_INFO.num_subcores
_RPW = _CHUNK // _NW
_LANES = 16


def _sc_route(w_hbm, ti_hbm, scores_hbm, w_v, ti_v, sc_v):
    wid = lax.axis_index("s") * _SC_INFO.num_cores + lax.axis_index("c")
    base = wid * _RPW

    pltpu.sync_copy(w_hbm.at[:, pl.ds(base, _RPW)], w_v)
    pltpu.sync_copy(ti_hbm.at[:, pl.ds(base, _RPW)], ti_v)

    zeros = jnp.zeros((_LANES,), jnp.float32)

    def _zero(i, _):
        sc_v[pl.ds(i * _LANES, _LANES)] = zeros
        return ()

    lax.fori_loop(0, _RPW * _E // _LANES, _zero, (), unroll=8)

    lane = lax.iota(jnp.int32, _LANES)

    def _group(g, _):
        sbase = (g * _LANES + lane) * _E
        for k in range(_K):
            wv = w_v[k, pl.ds(g * _LANES, _LANES)]
            ti = ti_v[k, pl.ds(g * _LANES, _LANES)]
            plsc.store_scatter(sc_v, [sbase + ti], wv)
        return ()

    lax.fori_loop(0, _RPW // _LANES, _group, (), unroll=2)

    pltpu.sync_copy(sc_v, scores_hbm.at[pl.ds(base * _E, _RPW * _E)])


def _sc_stage(wv, ti):
    mesh = plsc.VectorSubcoreMesh(core_axis_name="c", subcore_axis_name="s")
    fn = functools.partial(
        pl.kernel,
        mesh=mesh,
        compiler_params=pltpu.CompilerParams(needs_layout_passes=False),
        out_type=jax.ShapeDtypeStruct((_CHUNK * _E,), jnp.float32),
        scratch_types=[
            pltpu.VMEM((_K, _RPW), jnp.float32),
            pltpu.VMEM((_K, _RPW), jnp.int32),
            pltpu.VMEM((_RPW * _E,), jnp.float32),
        ],
    )(_sc_route)
    return fn(wv, ti)


def kernel(hidden_states, weight, bias):
    flat = hidden_states.reshape(-1, _EMBED)
    bias2d = bias.reshape(1, _E)
    scores_parts = []
    idx_parts = []
    for c in range(_CHUNKS):
        part = lax.slice_in_dim(flat, c * _CHUNK, (c + 1) * _CHUNK, axis=0)
        wv, ti, idx = _tc_logits_topk(part, weight, bias2d)
        scores_parts.append(_sc_stage(wv, ti).reshape(_CHUNK, _E))
        idx_parts.append(idx)
    return (
        jnp.concatenate(scores_parts, axis=0),
        jnp.concatenate(idx_parts, axis=0),
    )


# speedup vs baseline: 2.1892x; 2.1892x over previous
import functools

import jax
import jax.numpy as jnp
from jax import lax
from jax.experimental import pallas as pl
from jax.experimental.pallas import tpu as pltpu
from jax.experimental.pallas import tpu_sc as plsc

_EMBED = 4096
_E = 64
_K = 8
_ROWS = 1024
_N_ROWS = 16384
_CHUNKS = 2
_CHUNK = _N_ROWS // _CHUNKS


def _logits_topk_block(x_ref, w_ref, b_ref, wv_ref, ti_ref, idx_ref):
    x = x_ref[...]
    w = w_ref[...]
    logits = jax.lax.dot_general(
        x, w, (((1,), (1,)), ((), ())), preferred_element_type=jnp.float32
    ) + b_ref[...]

    lt = logits.T
    rows = jax.lax.broadcasted_iota(jnp.int32, lt.shape, 0)
    vals = lt
    maxes = []
    idxs = []
    for _ in range(_K):
        m = jnp.max(vals, axis=0, keepdims=True)
        a = jnp.min(jnp.where(vals == m, rows, _E), axis=0, keepdims=True)
        maxes.append(m)
        idxs.append(a)
        vals = jnp.where(rows == a, -jnp.inf, vals)

    e = [jnp.ones_like(maxes[0])] + [jnp.exp(m - maxes[0]) for m in maxes[1:]]
    denom = functools.reduce(jnp.add, e)
    ids = jnp.concatenate(idxs, axis=0)
    wv_ref[...] = jnp.concatenate(e, axis=0) / denom
    ti_ref[...] = ids
    idx_ref[...] = ids.T


def _tc_logits_topk(flat, weight, bias2d, chunk_idx):
    blocks = _CHUNK // _ROWS
    off = chunk_idx * blocks
    return pl.pallas_call(
        _logits_topk_block,
        grid=(blocks,),
        in_specs=[
            pl.BlockSpec((_ROWS, _EMBED), lambda i: (off + i, 0)),
            pl.BlockSpec((_E, _EMBED), lambda i: (0, 0)),
            pl.BlockSpec((1, _E), lambda i: (0, 0)),
        ],
        out_specs=[
            pl.BlockSpec((_K, _ROWS), lambda i: (0, i)),
            pl.BlockSpec((_K, _ROWS), lambda i: (0, i)),
            pl.BlockSpec((_ROWS, _K), lambda i: (i, 0)),
        ],
        out_shape=[
            jax.ShapeDtypeStruct((_K, _CHUNK), jnp.float32),
            jax.ShapeDtypeStruct((_K, _CHUNK), jnp.int32),
            jax.ShapeDtypeStruct((_CHUNK, _K), jnp.int32),
        ],
    )(flat, weight, bias2d)


_SC_INFO = plsc.get_sparse_core_info()
_NW = _SC_INFO.num_cores * _SC_INFO.num_subcores
_RPW = _CHUNK // _NW
_LANES = 16


def _sc_route(w_hbm, ti_hbm, scores_hbm, w_v, ti_v, sc_v):
    wid = lax.axis_index("s") * _SC_INFO.num_cores + lax.axis_index("c")
    base = wid * _RPW

    pltpu.sync_copy(w_hbm.at[:, pl.ds(base, _RPW)], w_v)
    pltpu.sync_copy(ti_hbm.at[:, pl.ds(base, _RPW)], ti_v)

    zeros = jnp.zeros((_LANES,), jnp.float32)

    def _zero(i, _):
        sc_v[pl.ds(i * _LANES, _LANES)] = zeros
        return ()

    lax.fori_loop(0, _RPW * _E // _LANES, _zero, (), unroll=8)

    lane = lax.iota(jnp.int32, _LANES)

    def _group(g, _):
        sbase = (g * _LANES + lane) * _E
        for k in range(_K):
            wv = w_v[k, pl.ds(g * _LANES, _LANES)]
            ti = ti_v[k, pl.ds(g * _LANES, _LANES)]
            plsc.store_scatter(sc_v, [sbase + ti], wv)
        return ()

    lax.fori_loop(0, _RPW // _LANES, _group, (), unroll=2)

    pltpu.sync_copy(sc_v, scores_hbm.at[pl.ds(base * _E, _RPW * _E)])


def _sc_stage(wv, ti):
    mesh = plsc.VectorSubcoreMesh(core_axis_name="c", subcore_axis_name="s")
    fn = functools.partial(
        pl.kernel,
        mesh=mesh,
        compiler_params=pltpu.CompilerParams(needs_layout_passes=False),
        out_type=jax.ShapeDtypeStruct((_CHUNK * _E,), jnp.float32),
        scratch_types=[
            pltpu.VMEM((_K, _RPW), jnp.float32),
            pltpu.VMEM((_K, _RPW), jnp.int32),
            pltpu.VMEM((_RPW * _E,), jnp.float32),
        ],
    )(_sc_route)
    return fn(wv, ti)


def kernel(hidden_states, weight, bias):
    flat = hidden_states.reshape(-1, _EMBED)
    bias2d = bias.reshape(1, _E)
    scores_parts = []
    idx_parts = []
    for c in range(_CHUNKS):
        wv, ti, idx = _tc_logits_topk(flat, weight, bias2d, c)
        scores_parts.append(_sc_stage(wv, ti).reshape(_CHUNK, _E))
        idx_parts.append(idx)
    return (
        jnp.concatenate(scores_parts, axis=0),
        jnp.concatenate(idx_parts, axis=0),
    )


# speedup vs baseline: 2.4293x; 1.1097x over previous
import functools

import jax
import jax.numpy as jnp
from jax import lax
from jax.experimental import pallas as pl
from jax.experimental.pallas import tpu as pltpu
from jax.experimental.pallas import tpu_sc as plsc

_EMBED = 4096
_E = 64
_K = 8
_ROWS = 1024
_N_ROWS = 16384
_CHUNKS = 1
_CHUNK = _N_ROWS // _CHUNKS


def _logits_topk_block(x_ref, w_ref, b_ref, wv_ref, ti_ref, idx_ref):
    x = x_ref[...]
    w = w_ref[...]
    logits = jax.lax.dot_general(
        x, w, (((1,), (1,)), ((), ())), preferred_element_type=jnp.float32
    ) + b_ref[...]

    lt = logits.T
    rows = jax.lax.broadcasted_iota(jnp.int32, lt.shape, 0)
    vals = lt
    maxes = []
    idxs = []
    for _ in range(_K):
        m = jnp.max(vals, axis=0, keepdims=True)
        a = jnp.min(jnp.where(vals == m, rows, _E), axis=0, keepdims=True)
        maxes.append(m)
        idxs.append(a)
        vals = jnp.where(rows == a, -jnp.inf, vals)

    e = [jnp.ones_like(maxes[0])] + [jnp.exp(m - maxes[0]) for m in maxes[1:]]
    denom = functools.reduce(jnp.add, e)
    ids = jnp.concatenate(idxs, axis=0)
    wv_ref[...] = jnp.concatenate(e, axis=0) / denom
    ti_ref[...] = ids
    idx_ref[...] = ids.T


def _tc_logits_topk(flat, weight, bias2d, chunk_idx):
    blocks = _CHUNK // _ROWS
    off = chunk_idx * blocks
    return pl.pallas_call(
        _logits_topk_block,
        grid=(blocks,),
        in_specs=[
            pl.BlockSpec((_ROWS, _EMBED), lambda i: (off + i, 0)),
            pl.BlockSpec((_E, _EMBED), lambda i: (0, 0)),
            pl.BlockSpec((1, _E), lambda i: (0, 0)),
        ],
        out_specs=[
            pl.BlockSpec((_K, _ROWS), lambda i: (0, i)),
            pl.BlockSpec((_K, _ROWS), lambda i: (0, i)),
            pl.BlockSpec((_ROWS, _K), lambda i: (i, 0)),
        ],
        out_shape=[
            jax.ShapeDtypeStruct((_K, _CHUNK), jnp.float32),
            jax.ShapeDtypeStruct((_K, _CHUNK), jnp.int32),
            jax.ShapeDtypeStruct((_CHUNK, _K), jnp.int32),
        ],
    )(flat, weight, bias2d)


_SC_INFO = plsc.get_sparse_core_info()
_NW = _SC_INFO.num_cores * _SC_INFO.num_subcores
_RPW = _CHUNK // _NW
_LANES = 16


def _sc_route(w_hbm, ti_hbm, scores_hbm, w_v, ti_v, sc_v):
    wid = lax.axis_index("s") * _SC_INFO.num_cores + lax.axis_index("c")
    base = wid * _RPW

    pltpu.sync_copy(w_hbm.at[:, pl.ds(base, _RPW)], w_v)
    pltpu.sync_copy(ti_hbm.at[:, pl.ds(base, _RPW)], ti_v)

    zeros = jnp.zeros((_LANES,), jnp.float32)

    def _zero(i, _):
        sc_v[pl.ds(i * _LANES, _LANES)] = zeros
        return ()

    lax.fori_loop(0, _RPW * _E // _LANES, _zero, (), unroll=8)

    lane = lax.iota(jnp.int32, _LANES)

    def _group(g, _):
        sbase = (g * _LANES + lane) * _E
        for k in range(_K):
            wv = w_v[k, pl.ds(g * _LANES, _LANES)]
            ti = ti_v[k, pl.ds(g * _LANES, _LANES)]
            plsc.store_scatter(sc_v, [sbase + ti], wv)
        return ()

    lax.fori_loop(0, _RPW // _LANES, _group, (), unroll=2)

    pltpu.sync_copy(sc_v, scores_hbm.at[pl.ds(base * _E, _RPW * _E)])


def _sc_stage(wv, ti):
    mesh = plsc.VectorSubcoreMesh(core_axis_name="c", subcore_axis_name="s")
    fn = functools.partial(
        pl.kernel,
        mesh=mesh,
        compiler_params=pltpu.CompilerParams(needs_layout_passes=False),
        out_type=jax.ShapeDtypeStruct((_CHUNK * _E,), jnp.float32),
        scratch_types=[
            pltpu.VMEM((_K, _RPW), jnp.float32),
            pltpu.VMEM((_K, _RPW), jnp.int32),
            pltpu.VMEM((_RPW * _E,), jnp.float32),
        ],
    )(_sc_route)
    return fn(wv, ti)


def kernel(hidden_states, weight, bias):
    flat = hidden_states.reshape(-1, _EMBED)
    bias2d = bias.reshape(1, _E)
    scores_parts = []
    idx_parts = []
    for c in range(_CHUNKS):
        wv, ti, idx = _tc_logits_topk(flat, weight, bias2d, c)
        scores_parts.append(_sc_stage(wv, ti).reshape(_CHUNK, _E))
        idx_parts.append(idx)
    return (
        jnp.concatenate(scores_parts, axis=0),
        jnp.concatenate(idx_parts, axis=0),
    )


# speedup vs baseline: 2.4621x; 1.0135x over previous
import functools

import jax
import jax.numpy as jnp
from jax import lax
from jax.experimental import pallas as pl
from jax.experimental.pallas import tpu as pltpu
from jax.experimental.pallas import tpu_sc as plsc

_EMBED = 4096
_E = 64
_K = 8
_ROWS = 1024
_N_ROWS = 16384
_CHUNKS = 1
_CHUNK = _N_ROWS // _CHUNKS


def _logits_topk_block(x_ref, w_ref, b_ref, wv_ref, ti_ref, idx_ref):
    x = x_ref[...]
    w = w_ref[...]
    logits = jax.lax.dot_general(
        x, w, (((1,), (1,)), ((), ())), preferred_element_type=jnp.float32
    ) + b_ref[...]

    lt = logits.T
    rows = jax.lax.broadcasted_iota(jnp.int32, lt.shape, 0)
    vals = lt
    maxes = []
    idxs = []
    for _ in range(_K):
        m = jnp.max(vals, axis=0, keepdims=True)
        a = jnp.min(jnp.where(vals == m, rows, _E), axis=0, keepdims=True)
        maxes.append(m)
        idxs.append(a)
        vals = jnp.where(rows == a, -jnp.inf, vals)

    e = [jnp.ones_like(maxes[0])] + [jnp.exp(m - maxes[0]) for m in maxes[1:]]
    denom = functools.reduce(jnp.add, e)
    ids = jnp.concatenate(idxs, axis=0)
    wv_ref[...] = jnp.concatenate(e, axis=0) / denom
    ti_ref[...] = ids
    idx_ref[...] = ids.T


def _tc_logits_topk(flat, weight, bias2d, chunk_idx):
    blocks = _CHUNK // _ROWS
    off = chunk_idx * blocks
    return pl.pallas_call(
        _logits_topk_block,
        grid=(blocks,),
        in_specs=[
            pl.BlockSpec((_ROWS, _EMBED), lambda i: (off + i, 0)),
            pl.BlockSpec((_E, _EMBED), lambda i: (0, 0)),
            pl.BlockSpec((1, _E), lambda i: (0, 0)),
        ],
        out_specs=[
            pl.BlockSpec((_K, _ROWS), lambda i: (0, i)),
            pl.BlockSpec((_K, _ROWS), lambda i: (0, i)),
            pl.BlockSpec((_ROWS, _K), lambda i: (i, 0)),
        ],
        out_shape=[
            jax.ShapeDtypeStruct((_K, _CHUNK), jnp.float32),
            jax.ShapeDtypeStruct((_K, _CHUNK), jnp.int32),
            jax.ShapeDtypeStruct((_CHUNK, _K), jnp.int32),
        ],
    )(flat, weight, bias2d)


_SC_INFO = plsc.get_sparse_core_info()
_NW = _SC_INFO.num_cores * _SC_INFO.num_subcores
_RPW = _CHUNK // _NW
_LANES = 16


def _sc_route(w_hbm, ti_hbm, scores_hbm, w_v, ti_v, sc_v, sem):
    wid = lax.axis_index("s") * _SC_INFO.num_cores + lax.axis_index("c")
    base = wid * _RPW

    cp_w = pltpu.async_copy(w_hbm.at[:, pl.ds(base, _RPW)], w_v, sem)
    cp_t = pltpu.async_copy(ti_hbm.at[:, pl.ds(base, _RPW)], ti_v, sem)

    zeros = jnp.zeros((_LANES,), jnp.float32)

    def _zero(i, _):
        sc_v[pl.ds(i * _LANES, _LANES)] = zeros
        return ()

    lax.fori_loop(0, _RPW * _E // _LANES, _zero, (), unroll=8)
    cp_w.wait()
    cp_t.wait()

    lane = lax.iota(jnp.int32, _LANES)

    def _group(g, _):
        sbase = (g * _LANES + lane) * _E
        for k in range(_K):
            wv = w_v[k, pl.ds(g * _LANES, _LANES)]
            ti = ti_v[k, pl.ds(g * _LANES, _LANES)]
            plsc.store_scatter(sc_v, [sbase + ti], wv)
        return ()

    lax.fori_loop(0, _RPW // _LANES, _group, (), unroll=2)

    pltpu.sync_copy(sc_v, scores_hbm.at[pl.ds(base * _E, _RPW * _E)])


def _sc_stage(wv, ti):
    mesh = plsc.VectorSubcoreMesh(core_axis_name="c", subcore_axis_name="s")
    fn = functools.partial(
        pl.kernel,
        mesh=mesh,
        compiler_params=pltpu.CompilerParams(needs_layout_passes=False),
        out_type=jax.ShapeDtypeStruct((_CHUNK * _E,), jnp.float32),
        scratch_types=[
            pltpu.VMEM((_K, _RPW), jnp.float32),
            pltpu.VMEM((_K, _RPW), jnp.int32),
            pltpu.VMEM((_RPW * _E,), jnp.float32),
            pltpu.SemaphoreType.DMA,
        ],
    )(_sc_route)
    return fn(wv, ti)


def kernel(hidden_states, weight, bias):
    flat = hidden_states.reshape(-1, _EMBED)
    bias2d = bias.reshape(1, _E)
    scores_parts = []
    idx_parts = []
    for c in range(_CHUNKS):
        wv, ti, idx = _tc_logits_topk(flat, weight, bias2d, c)
        scores_parts.append(_sc_stage(wv, ti).reshape(_CHUNK, _E))
        idx_parts.append(idx)
    if _CHUNKS == 1:
        return (scores_parts[0], idx_parts[0])
    return (
        jnp.concatenate(scores_parts, axis=0),
        jnp.concatenate(idx_parts, axis=0),
    )
